# Initial kernel scaffold; baseline (speedup 1.0000x reference)
import jax
import jax.numpy as jnp
from jax.experimental import pallas as pl


def kernel(x, edge_index, edge_attr, batch, params):
    raise NotImplementedError("write your pallas kernel here")


# speedup vs baseline: 2.6859x; 2.6859x over previous
import functools

import jax
import jax.numpy as jnp
from jax import lax
from jax.experimental import pallas as pl
from jax.experimental.pallas import tpu as pltpu
from jax.experimental.pallas import tpu_sc as plsc

F32 = jnp.float32

_N = 10000
_E = 320000
_H = 128
_G = 256

_NC = 2
_NS = 16
_NW = _NC * _NS

_C = 80
_EPW = _E // _NW
_NCHUNK = _EPW // _C
_NPAD = 10240
_RPT = _NPAD // _NS

_BLK = 1000
_NB = _N // _BLK
_EBLK = 4000
_NEB = _E // _EBLK


def _mp_body(h_hbm, e_hbm, src_hbm, dst_hbm, out_hbm, sidx, didx, hrows, erows,
             agg, sem):
    c = lax.axis_index("c")
    s = lax.axis_index("s")
    wid = c * _NS + s

    def zrow(i, carry):
        for j in range(_H // 16):
            hrows[i, pl.ds(j * 16, 16)] = jnp.zeros((16,), F32)
        return carry
    lax.fori_loop(0, _C, zrow, 0)

    base = s * _RPT
    n_full = _RPT // _C

    def zcopy(k, carry):
        pltpu.sync_copy(hrows, agg.at[pl.ds(base + k * _C, _C)])
        return carry
    lax.fori_loop(0, n_full, zcopy, 0)
    plsc.subcore_barrier()

    ebase = wid * _EPW

    def chunk(k, carry):
        off = ebase + k * _C
        pltpu.sync_copy(src_hbm.at[pl.ds(off, _C)], sidx)
        pltpu.sync_copy(dst_hbm.at[pl.ds(off, _C)], didx)
        pltpu.async_copy(h_hbm.at[sidx], hrows, sem).wait()
        pltpu.sync_copy(e_hbm.at[pl.ds(off, _C)], erows)

        def crow(i, inner):
            for j in range(_H // 16):
                sl = pl.ds(j * 16, 16)
                hrows[i, sl] = jnp.maximum(hrows[i, sl] + erows[i, sl], 0.0)
            return inner
        lax.fori_loop(0, _C, crow, 0)

        pltpu.sync_copy(hrows, agg.at[didx], add=True)
        return carry
    lax.fori_loop(0, _NCHUNK, chunk, 0)

    plsc.subcore_barrier()
    pltpu.sync_copy(agg.at[pl.ds(base, _RPT)],
                    out_hbm.at[c, pl.ds(base, _RPT)])


@functools.cache
def _mp_kernel():
    return functools.partial(
        pl.kernel,
        out_type=jax.ShapeDtypeStruct((_NC, _NPAD, _H), F32),
        mesh=plsc.VectorSubcoreMesh(core_axis_name="c", subcore_axis_name="s",
                                    num_cores=_NC, num_subcores=_NS),
        scratch_types=[
            pltpu.VMEM((_C,), jnp.int32),
            pltpu.VMEM((_C,), jnp.int32),
            pltpu.VMEM((_C, _H), F32),
            pltpu.VMEM((_C, _H), F32),
            pltpu.VMEM_SHARED((_NPAD, _H), F32),
            pltpu.SemaphoreType.DMA,
        ],
    )(_mp_body)


def _mp(h, e, src, dst):
    return _mp_kernel()(h, e, src, dst)


def _enc_body(x_ref, w_ref, b_ref, o_ref):
    o_ref[...] = (
        jnp.dot(x_ref[...], w_ref[...], preferred_element_type=F32)
        + b_ref[...]
    )


def _enc(x, w, b):
    n, d = x.shape
    h = w.shape[1]
    blk = _BLK if n == _N else _EBLK
    return pl.pallas_call(
        _enc_body,
        grid=(n // blk,),
        in_specs=[
            pl.BlockSpec((blk, d), lambda i: (i, 0)),
            pl.BlockSpec((d, h), lambda i: (0, 0)),
            pl.BlockSpec((1, h), lambda i: (0, 0)),
        ],
        out_specs=pl.BlockSpec((blk, h), lambda i: (i, 0)),
        out_shape=jax.ShapeDtypeStruct((n, h), F32),
    )(x, w, b)


def _mlp1_body(s_ref, h_ref, a_ref, w_ref, b_ref, y_ref, st_ref):
    i = pl.program_id(0)
    z = s_ref[0] * h_ref[...] + a_ref[0] + a_ref[1]
    y = jnp.dot(z, w_ref[...], preferred_element_type=F32) + b_ref[...]
    y_ref[...] = y

    @pl.when(i == 0)
    def _():
        st_ref[...] = jnp.zeros_like(st_ref)

    st_ref[0:1, :] += jnp.sum(y, axis=0, keepdims=True)
    st_ref[1:2, :] += jnp.sum(y * y, axis=0, keepdims=True)


def _mlp1(scale, h, agg, w1, b1):
    f = w1.shape[1]
    return pl.pallas_call(
        _mlp1_body,
        grid=(_NB,),
        in_specs=[
            pl.BlockSpec(memory_space=pltpu.SMEM),
            pl.BlockSpec((_BLK, _H), lambda i: (i, 0)),
            pl.BlockSpec((_NC, _BLK, _H), lambda i: (0, i, 0)),
            pl.BlockSpec((_H, f), lambda i: (0, 0)),
            pl.BlockSpec((1, f), lambda i: (0, 0)),
        ],
        out_specs=[
            pl.BlockSpec((_BLK, f), lambda i: (i, 0)),
            pl.BlockSpec((8, f), lambda i: (0, 0)),
        ],
        out_shape=[
            jax.ShapeDtypeStruct((_N, f), F32),
            jax.ShapeDtypeStruct((8, f), F32),
        ],
    )(scale, h, agg, w1, b1)


def _mlp2_body(st_ref, g_ref, bt_ref, y1_ref, w_ref, b_ref, y2_ref, st2_ref):
    i = pl.program_id(0)
    mean = st_ref[0:1, :] * (1.0 / _N)
    var = st_ref[1:2, :] * (1.0 / _N) - mean * mean
    scale = lax.rsqrt(var + 1e-5) * g_ref[...]
    r = jnp.maximum((y1_ref[...] - mean) * scale + bt_ref[...], 0.0)
    y2 = jnp.dot(r, w_ref[...], preferred_element_type=F32) + b_ref[...]
    y2_ref[...] = y2

    @pl.when(i == 0)
    def _():
        st2_ref[...] = jnp.zeros_like(st2_ref)

    st2_ref[0:1, :] += jnp.sum(y2, axis=0, keepdims=True)
    st2_ref[1:2, :] += jnp.sum(y2 * y2, axis=0, keepdims=True)


def _mlp2(st, g, bt, y1, w2, b2):
    f1 = y1.shape[1]
    f2 = w2.shape[1]
    return pl.pallas_call(
        _mlp2_body,
        grid=(_NB,),
        in_specs=[
            pl.BlockSpec((8, f1), lambda i: (0, 0)),
            pl.BlockSpec((1, f1), lambda i: (0, 0)),
            pl.BlockSpec((1, f1), lambda i: (0, 0)),
            pl.BlockSpec((_BLK, f1), lambda i: (i, 0)),
            pl.BlockSpec((f1, f2), lambda i: (0, 0)),
            pl.BlockSpec((1, f2), lambda i: (0, 0)),
        ],
        out_specs=[
            pl.BlockSpec((_BLK, f2), lambda i: (i, 0)),
            pl.BlockSpec((8, f2), lambda i: (0, 0)),
        ],
        out_shape=[
            jax.ShapeDtypeStruct((_N, f2), F32),
            jax.ShapeDtypeStruct((8, f2), F32),
        ],
    )(st, g, bt, y1, w2, b2)


def _mlp3_body(st_ref, g_ref, bt_ref, y_ref, o_ref):
    mean = st_ref[0:1, :] * (1.0 / _N)
    var = st_ref[1:2, :] * (1.0 / _N) - mean * mean
    scale = lax.rsqrt(var + 1e-5) * g_ref[...]
    o_ref[...] = jnp.maximum((y_ref[...] - mean) * scale + bt_ref[...], 0.0)


def _mlp3(st, g, bt, y):
    f = y.shape[1]
    return pl.pallas_call(
        _mlp3_body,
        grid=(_NB,),
        in_specs=[
            pl.BlockSpec((8, f), lambda i: (0, 0)),
            pl.BlockSpec((1, f), lambda i: (0, 0)),
            pl.BlockSpec((1, f), lambda i: (0, 0)),
            pl.BlockSpec((_BLK, f), lambda i: (i, 0)),
        ],
        out_specs=pl.BlockSpec((_BLK, f), lambda i: (i, 0)),
        out_shape=jax.ShapeDtypeStruct((_N, f), F32),
    )(st, g, bt, y)


def _pool_body(b_ref, h0_ref, h1_ref, h2_ref, h3_ref, w1_ref, b1_ref, g_ref,
               bt_ref, wo_ref, bo_ref, o_ref, acc_ref, cnt_ref):
    i = pl.program_id(0)

    @pl.when(i == 0)
    def _():
        acc_ref[...] = jnp.zeros_like(acc_ref)
        cnt_ref[...] = jnp.zeros_like(cnt_ref)

    b = b_ref[0, 0, :]
    oh = (b[:, None] == lax.broadcasted_iota(jnp.int32, (_BLK, _G), 1))
    oh = oh.astype(F32)
    hj = jnp.concatenate(
        [h0_ref[...], h1_ref[...], h2_ref[...], h3_ref[...]], axis=1)
    acc_ref[...] += lax.dot_general(
        oh, hj, (((0,), (0,)), ((), ())), preferred_element_type=F32,
        precision=lax.Precision.HIGHEST)
    cnt_ref[...] += lax.dot_general(
        oh, jnp.ones((_BLK, 8), F32), (((0,), (0,)), ((), ())),
        preferred_element_type=F32, precision=lax.Precision.HIGHEST)

    @pl.when(i == _NB - 1)
    def _():
        cnt = jnp.maximum(cnt_ref[:, 0:1], 1.0)
        pooled = acc_ref[...] / cnt
        o1 = jnp.dot(pooled, w1_ref[...], preferred_element_type=F32)
        o1 = o1 + b1_ref[...]
        mean = jnp.mean(o1, axis=0, keepdims=True)
        var = jnp.mean(o1 * o1, axis=0, keepdims=True) - mean * mean
        scale = lax.rsqrt(var + 1e-5) * g_ref[...]
        r = jnp.maximum((o1 - mean) * scale + bt_ref[...], 0.0)
        o_ref[...] = (
            jnp.dot(r, wo_ref[...], preferred_element_type=F32) + bo_ref[...]
        )


def _pool_cls(batch3, reps, w1, b1, g, bt, wo, bo):
    jk = w1.shape[0]
    ffn = w1.shape[1]
    out = wo.shape[1]
    return pl.pallas_call(
        _pool_body,
        grid=(_NB,),
        in_specs=[
            pl.BlockSpec((1, 1, _BLK), lambda i: (i, 0, 0)),
            pl.BlockSpec((_BLK, _H), lambda i: (i, 0)),
            pl.BlockSpec((_BLK, _H), lambda i: (i, 0)),
            pl.BlockSpec((_BLK, _H), lambda i: (i, 0)),
            pl.BlockSpec((_BLK, _H), lambda i: (i, 0)),
            pl.BlockSpec((jk, ffn), lambda i: (0, 0)),
            pl.BlockSpec((1, ffn), lambda i: (0, 0)),
            pl.BlockSpec((1, ffn), lambda i: (0, 0)),
            pl.BlockSpec((1, ffn), lambda i: (0, 0)),
            pl.BlockSpec((ffn, out), lambda i: (0, 0)),
            pl.BlockSpec((1, out), lambda i: (0, 0)),
        ],
        out_specs=pl.BlockSpec((_G, out), lambda i: (0, 0)),
        out_shape=jax.ShapeDtypeStruct((_G, out), F32),
        scratch_shapes=[
            pltpu.VMEM((_G, jk), F32),
            pltpu.VMEM((_G, 8), F32),
        ],
    )(batch3, *reps, w1, b1, g, bt, wo, bo)


def kernel(x, edge_index, edge_attr, batch, params):
    src = edge_index[0]
    dst = edge_index[1]

    h = _enc(x, params['enc_W'], params['enc_b'].reshape(1, -1))
    e = _enc(edge_attr, params['edge_W'], params['edge_b'].reshape(1, -1))

    reps = [h]
    for l in range(3):
        agg = _mp(h, e, src, dst)
        scale = (1.0 + params[f'conv{l}_eps']).reshape(1)
        y1, st1 = _mlp1(scale, h, agg, params[f'conv{l}_W1'],
                        params[f'conv{l}_b1'].reshape(1, -1))
        y2, st2 = _mlp2(st1, params[f'conv{l}_g1'].reshape(1, -1),
                        params[f'conv{l}_bt1'].reshape(1, -1), y1,
                        params[f'conv{l}_W2'],
                        params[f'conv{l}_b2'].reshape(1, -1))
        h = _mlp3(st2, params[f'conv{l}_g2'].reshape(1, -1),
                  params[f'conv{l}_bt2'].reshape(1, -1), y2)
        reps.append(h)

    batch3 = batch.reshape(_NB, 1, _BLK)
    return _pool_cls(batch3, reps, params['cls_W1'],
                     params['cls_b1'].reshape(1, -1),
                     params['cls_g'].reshape(1, -1),
                     params['cls_bt'].reshape(1, -1),
                     params['out_W'], params['out_b'].reshape(1, -1))


# speedup vs baseline: 5.1418x; 1.9144x over previous
import functools

import jax
import jax.numpy as jnp
from jax import lax
from jax.experimental import pallas as pl
from jax.experimental.pallas import tpu as pltpu
from jax.experimental.pallas import tpu_sc as plsc

F32 = jnp.float32

_N = 10000
_E = 320000
_H = 128
_G = 256

_NC = 2
_NS = 16
_NW = _NC * _NS

_C = 40
_EPW = _E // _NW
_NCHUNK = _EPW // _C
_NPAD = 10240
_RPT = _NPAD // _NS

_BLK = 1000
_NB = _N // _BLK
_EBLK = 4000
_NEB = _E // _EBLK


def _mp_body(h_hbm, e_hbm, src_hbm, dst_hbm, out_hbm,
             h0, h1, h2, h3, e0, e1, e2, e3,
             si0, si1, si2, si3, di0, di1, di2, di3,
             agg,
             sgs, ses, sis, sds, sss):
    hb = [h0, h1, h2, h3]
    eb = [e0, e1, e2, e3]
    sib = [si0, si1, si2, si3]
    dib = [di0, di1, di2, di3]

    c = lax.axis_index("c")
    s = lax.axis_index("s")
    wid = c * _NS + s
    ebase = wid * _EPW
    last = _NCHUNK - 1

    def zrow(i, carry):
        for j in range(_H // 16):
            h0[i, pl.ds(j * 16, 16)] = jnp.zeros((16,), F32)
        return carry
    lax.fori_loop(0, _C, zrow, 0)

    base = s * _RPT

    def zcopy(k, carry):
        pltpu.sync_copy(h0, agg.at[pl.ds(base + k * _C, _C)])
        return carry
    lax.fori_loop(0, _RPT // _C, zcopy, 0)
    plsc.subcore_barrier()

    def fetch_ie(k, b):
        pltpu.async_copy(src_hbm.at[pl.ds(ebase + k * _C, _C)], sib[b],
                         sis.at[b])
        pltpu.async_copy(dst_hbm.at[pl.ds(ebase + k * _C, _C)], dib[b],
                         sds.at[b])
        pltpu.async_copy(e_hbm.at[pl.ds(ebase + k * _C, _C)], eb[b],
                         ses.at[b])

    def wait_si(b):
        pltpu.make_async_copy(src_hbm.at[pl.ds(ebase, _C)], sib[b],
                              sis.at[b]).wait()

    def start_gather(b):
        pltpu.async_copy(h_hbm.at[sib[b]], hb[b], sgs.at[b])

    def wait_gather_ed(b):
        pltpu.make_async_copy(h_hbm.at[sib[b]], hb[b], sgs.at[b]).wait()
        pltpu.make_async_copy(e_hbm.at[pl.ds(ebase, _C)], eb[b],
                              ses.at[b]).wait()
        pltpu.make_async_copy(dst_hbm.at[pl.ds(ebase, _C)], dib[b],
                              sds.at[b]).wait()

    def compute(b):
        hbuf, ebuf = hb[b], eb[b]

        def crow(i, carry):
            for j in range(_H // 16):
                sl = pl.ds(j * 16, 16)
                hbuf[i, sl] = jnp.maximum(hbuf[i, sl] + ebuf[i, sl], 0.0)
            return carry
        lax.fori_loop(0, _C, crow, 0)

    def start_scatter(b):
        pltpu.async_copy(hb[b], agg.at[dib[b]], sss.at[b], add=True)

    def wait_scatter(b):
        pltpu.make_async_copy(hb[b], agg.at[dib[b]], sss.at[b]).wait()

    fetch_ie(0, 0)
    fetch_ie(1, 1)
    fetch_ie(2, 2)
    wait_si(0)
    start_gather(0)

    def quad(i, carry):
        for u in range(4):
            k = 4 * i + u
            b = u
            pb = (u + 3) % 4
            nb = (u + 1) % 4

            @pl.when(jnp.logical_and(k + 3 <= last, k >= 1))
            def _():
                wait_scatter(pb)
                fetch_ie(k + 3, pb)

            @pl.when(jnp.logical_and(k + 3 <= last, k == 0))
            def _():
                fetch_ie(k + 3, pb)

            @pl.when(k + 1 <= last)
            def _():
                wait_si(nb)
                start_gather(nb)

            @pl.when(k <= last)
            def _():
                wait_gather_ed(b)
                compute(b)
                start_scatter(b)
        return carry
    lax.fori_loop(0, (_NCHUNK + 3) // 4, quad, 0)

    for b in range(4):
        wait_scatter(b)
    plsc.subcore_barrier()
    pltpu.sync_copy(agg.at[pl.ds(base, _RPT)],
                    out_hbm.at[c, pl.ds(base, _RPT)])


@functools.cache
def _mp_kernel():
    return functools.partial(
        pl.kernel,
        out_type=jax.ShapeDtypeStruct((_NC, _NPAD, _H), F32),
        mesh=plsc.VectorSubcoreMesh(core_axis_name="c", subcore_axis_name="s",
                                    num_cores=_NC, num_subcores=_NS),
        scratch_types=(
            [pltpu.VMEM((_C, _H), F32)] * 8
            + [pltpu.VMEM((_C,), jnp.int32)] * 8
            + [pltpu.VMEM_SHARED((_NPAD, _H), F32)]
            + [pltpu.SemaphoreType.DMA((4,))] * 5
        ),
    )(_mp_body)


def _mp(h, e, src, dst):
    return _mp_kernel()(h, e, src, dst)


def _enc_body(x_ref, w_ref, b_ref, o_ref):
    o_ref[...] = (
        jnp.dot(x_ref[...], w_ref[...], preferred_element_type=F32)
        + b_ref[...]
    )


def _enc(x, w, b):
    n, d = x.shape
    h = w.shape[1]
    blk = _BLK if n == _N else _EBLK
    return pl.pallas_call(
        _enc_body,
        grid=(n // blk,),
        in_specs=[
            pl.BlockSpec((blk, d), lambda i: (i, 0)),
            pl.BlockSpec((d, h), lambda i: (0, 0)),
            pl.BlockSpec((1, h), lambda i: (0, 0)),
        ],
        out_specs=pl.BlockSpec((blk, h), lambda i: (i, 0)),
        out_shape=jax.ShapeDtypeStruct((n, h), F32),
    )(x, w, b)


def _mlp1_body(s_ref, h_ref, a_ref, w_ref, b_ref, y_ref, st_ref):
    i = pl.program_id(0)
    z = s_ref[0] * h_ref[...] + a_ref[0] + a_ref[1]
    y = jnp.dot(z, w_ref[...], preferred_element_type=F32) + b_ref[...]
    y_ref[...] = y

    @pl.when(i == 0)
    def _():
        st_ref[...] = jnp.zeros_like(st_ref)

    st_ref[0:1, :] += jnp.sum(y, axis=0, keepdims=True)
    st_ref[1:2, :] += jnp.sum(y * y, axis=0, keepdims=True)


def _mlp1(scale, h, agg, w1, b1):
    f = w1.shape[1]
    return pl.pallas_call(
        _mlp1_body,
        grid=(_NB,),
        in_specs=[
            pl.BlockSpec(memory_space=pltpu.SMEM),
            pl.BlockSpec((_BLK, _H), lambda i: (i, 0)),
            pl.BlockSpec((_NC, _BLK, _H), lambda i: (0, i, 0)),
            pl.BlockSpec((_H, f), lambda i: (0, 0)),
            pl.BlockSpec((1, f), lambda i: (0, 0)),
        ],
        out_specs=[
            pl.BlockSpec((_BLK, f), lambda i: (i, 0)),
            pl.BlockSpec((8, f), lambda i: (0, 0)),
        ],
        out_shape=[
            jax.ShapeDtypeStruct((_N, f), F32),
            jax.ShapeDtypeStruct((8, f), F32),
        ],
    )(scale, h, agg, w1, b1)


def _mlp2_body(st_ref, g_ref, bt_ref, y1_ref, w_ref, b_ref, y2_ref, st2_ref):
    i = pl.program_id(0)
    mean = st_ref[0:1, :] * (1.0 / _N)
    var = st_ref[1:2, :] * (1.0 / _N) - mean * mean
    scale = lax.rsqrt(var + 1e-5) * g_ref[...]
    r = jnp.maximum((y1_ref[...] - mean) * scale + bt_ref[...], 0.0)
    y2 = jnp.dot(r, w_ref[...], preferred_element_type=F32) + b_ref[...]
    y2_ref[...] = y2

    @pl.when(i == 0)
    def _():
        st2_ref[...] = jnp.zeros_like(st2_ref)

    st2_ref[0:1, :] += jnp.sum(y2, axis=0, keepdims=True)
    st2_ref[1:2, :] += jnp.sum(y2 * y2, axis=0, keepdims=True)


def _mlp2(st, g, bt, y1, w2, b2):
    f1 = y1.shape[1]
    f2 = w2.shape[1]
    return pl.pallas_call(
        _mlp2_body,
        grid=(_NB,),
        in_specs=[
            pl.BlockSpec((8, f1), lambda i: (0, 0)),
            pl.BlockSpec((1, f1), lambda i: (0, 0)),
            pl.BlockSpec((1, f1), lambda i: (0, 0)),
            pl.BlockSpec((_BLK, f1), lambda i: (i, 0)),
            pl.BlockSpec((f1, f2), lambda i: (0, 0)),
            pl.BlockSpec((1, f2), lambda i: (0, 0)),
        ],
        out_specs=[
            pl.BlockSpec((_BLK, f2), lambda i: (i, 0)),
            pl.BlockSpec((8, f2), lambda i: (0, 0)),
        ],
        out_shape=[
            jax.ShapeDtypeStruct((_N, f2), F32),
            jax.ShapeDtypeStruct((8, f2), F32),
        ],
    )(st, g, bt, y1, w2, b2)


def _mlp3_body(st_ref, g_ref, bt_ref, y_ref, o_ref):
    mean = st_ref[0:1, :] * (1.0 / _N)
    var = st_ref[1:2, :] * (1.0 / _N) - mean * mean
    scale = lax.rsqrt(var + 1e-5) * g_ref[...]
    o_ref[...] = jnp.maximum((y_ref[...] - mean) * scale + bt_ref[...], 0.0)


def _mlp3(st, g, bt, y):
    f = y.shape[1]
    return pl.pallas_call(
        _mlp3_body,
        grid=(_NB,),
        in_specs=[
            pl.BlockSpec((8, f), lambda i: (0, 0)),
            pl.BlockSpec((1, f), lambda i: (0, 0)),
            pl.BlockSpec((1, f), lambda i: (0, 0)),
            pl.BlockSpec((_BLK, f), lambda i: (i, 0)),
        ],
        out_specs=pl.BlockSpec((_BLK, f), lambda i: (i, 0)),
        out_shape=jax.ShapeDtypeStruct((_N, f), F32),
    )(st, g, bt, y)


def _pool_body(b_ref, h0_ref, h1_ref, h2_ref, h3_ref, w1_ref, b1_ref, g_ref,
               bt_ref, wo_ref, bo_ref, o_ref, acc_ref, cnt_ref):
    i = pl.program_id(0)

    @pl.when(i == 0)
    def _():
        acc_ref[...] = jnp.zeros_like(acc_ref)
        cnt_ref[...] = jnp.zeros_like(cnt_ref)

    b = b_ref[0, 0, :]
    oh = (b[:, None] == lax.broadcasted_iota(jnp.int32, (_BLK, _G), 1))
    oh = oh.astype(F32)
    hj = jnp.concatenate(
        [h0_ref[...], h1_ref[...], h2_ref[...], h3_ref[...]], axis=1)
    acc_ref[...] += lax.dot_general(
        oh, hj, (((0,), (0,)), ((), ())), preferred_element_type=F32,
        precision=lax.Precision.HIGHEST)
    cnt_ref[...] += lax.dot_general(
        oh, jnp.ones((_BLK, 8), F32), (((0,), (0,)), ((), ())),
        preferred_element_type=F32, precision=lax.Precision.HIGHEST)

    @pl.when(i == _NB - 1)
    def _():
        cnt = jnp.maximum(cnt_ref[:, 0:1], 1.0)
        pooled = acc_ref[...] / cnt
        o1 = jnp.dot(pooled, w1_ref[...], preferred_element_type=F32)
        o1 = o1 + b1_ref[...]
        mean = jnp.mean(o1, axis=0, keepdims=True)
        var = jnp.mean(o1 * o1, axis=0, keepdims=True) - mean * mean
        scale = lax.rsqrt(var + 1e-5) * g_ref[...]
        r = jnp.maximum((o1 - mean) * scale + bt_ref[...], 0.0)
        o_ref[...] = (
            jnp.dot(r, wo_ref[...], preferred_element_type=F32) + bo_ref[...]
        )


def _pool_cls(batch3, reps, w1, b1, g, bt, wo, bo):
    jk = w1.shape[0]
    ffn = w1.shape[1]
    out = wo.shape[1]
    return pl.pallas_call(
        _pool_body,
        grid=(_NB,),
        in_specs=[
            pl.BlockSpec((1, 1, _BLK), lambda i: (i, 0, 0)),
            pl.BlockSpec((_BLK, _H), lambda i: (i, 0)),
            pl.BlockSpec((_BLK, _H), lambda i: (i, 0)),
            pl.BlockSpec((_BLK, _H), lambda i: (i, 0)),
            pl.BlockSpec((_BLK, _H), lambda i: (i, 0)),
            pl.BlockSpec((jk, ffn), lambda i: (0, 0)),
            pl.BlockSpec((1, ffn), lambda i: (0, 0)),
            pl.BlockSpec((1, ffn), lambda i: (0, 0)),
            pl.BlockSpec((1, ffn), lambda i: (0, 0)),
            pl.BlockSpec((ffn, out), lambda i: (0, 0)),
            pl.BlockSpec((1, out), lambda i: (0, 0)),
        ],
        out_specs=pl.BlockSpec((_G, out), lambda i: (0, 0)),
        out_shape=jax.ShapeDtypeStruct((_G, out), F32),
        scratch_shapes=[
            pltpu.VMEM((_G, jk), F32),
            pltpu.VMEM((_G, 8), F32),
        ],
    )(batch3, *reps, w1, b1, g, bt, wo, bo)


def kernel(x, edge_index, edge_attr, batch, params):
    src = edge_index[0]
    dst = edge_index[1]

    h = _enc(x, params['enc_W'], params['enc_b'].reshape(1, -1))
    e = _enc(edge_attr, params['edge_W'], params['edge_b'].reshape(1, -1))

    reps = [h]
    for l in range(3):
        agg = _mp(h, e, src, dst)
        scale = (1.0 + params[f'conv{l}_eps']).reshape(1)
        y1, st1 = _mlp1(scale, h, agg, params[f'conv{l}_W1'],
                        params[f'conv{l}_b1'].reshape(1, -1))
        y2, st2 = _mlp2(st1, params[f'conv{l}_g1'].reshape(1, -1),
                        params[f'conv{l}_bt1'].reshape(1, -1), y1,
                        params[f'conv{l}_W2'],
                        params[f'conv{l}_b2'].reshape(1, -1))
        h = _mlp3(st2, params[f'conv{l}_g2'].reshape(1, -1),
                  params[f'conv{l}_bt2'].reshape(1, -1), y2)
        reps.append(h)

    batch3 = batch.reshape(_NB, 1, _BLK)
    return _pool_cls(batch3, reps, params['cls_W1'],
                     params['cls_b1'].reshape(1, -1),
                     params['cls_g'].reshape(1, -1),
                     params['cls_bt'].reshape(1, -1),
                     params['out_W'], params['out_b'].reshape(1, -1))


# speedup vs baseline: 5.2452x; 1.0201x over previous
import functools

import jax
import jax.numpy as jnp
from jax import lax
from jax.experimental import pallas as pl
from jax.experimental.pallas import tpu as pltpu
from jax.experimental.pallas import tpu_sc as plsc

F32 = jnp.float32

_N = 10000
_E = 320000
_H = 128
_G = 256

_NC = 2
_NS = 16
_NW = _NC * _NS

_C = 40
_EPW = _E // _NW
_NCHUNK = _EPW // _C
_NPAD = 10240
_RPT = _NPAD // _NS

_BLK = 1000
_NB = _N // _BLK
_EBLK = 4000
_NEB = _E // _EBLK


def _mp_body(h_hbm, e_hbm, src_hbm, dst_hbm, out_hbm,
             h0, h1, h2, h3, e0, e1, e2, e3,
             si0, si1, si2, si3, di0, di1, di2, di3,
             agg,
             sgs, ses, sis, sds, sss):
    hb = [h0, h1, h2, h3]
    eb = [e0, e1, e2, e3]
    sib = [si0, si1, si2, si3]
    dib = [di0, di1, di2, di3]

    c = lax.axis_index("c")
    s = lax.axis_index("s")
    wid = c * _NS + s
    ebase = wid * _EPW
    last = _NCHUNK - 1

    def zrow(i, carry):
        for j in range(_H // 16):
            h0[i, pl.ds(j * 16, 16)] = jnp.zeros((16,), F32)
        return carry
    lax.fori_loop(0, _C, zrow, 0)

    base = s * _RPT

    def zcopy(k, carry):
        pltpu.sync_copy(h0, agg.at[pl.ds(base + k * _C, _C)])
        return carry
    lax.fori_loop(0, _RPT // _C, zcopy, 0)
    plsc.subcore_barrier()

    def fetch_ie(k, b):
        pltpu.async_copy(src_hbm.at[pl.ds(ebase + k * _C, _C)], sib[b],
                         sis.at[b])
        pltpu.async_copy(dst_hbm.at[pl.ds(ebase + k * _C, _C)], dib[b],
                         sds.at[b])
        pltpu.async_copy(e_hbm.at[pl.ds(ebase + k * _C, _C)], eb[b],
                         ses.at[b])

    def wait_si(b):
        pltpu.make_async_copy(src_hbm.at[pl.ds(ebase, _C)], sib[b],
                              sis.at[b]).wait()

    def start_gather(b):
        pltpu.async_copy(h_hbm.at[sib[b]], hb[b], sgs.at[b])

    def wait_gather_ed(b):
        pltpu.make_async_copy(h_hbm.at[sib[b]], hb[b], sgs.at[b]).wait()
        pltpu.make_async_copy(e_hbm.at[pl.ds(ebase, _C)], eb[b],
                              ses.at[b]).wait()
        pltpu.make_async_copy(dst_hbm.at[pl.ds(ebase, _C)], dib[b],
                              sds.at[b]).wait()

    def compute(b):
        hbuf, ebuf = hb[b], eb[b]

        def crow(i, carry):
            for j in range(_H // 16):
                sl = pl.ds(j * 16, 16)
                hbuf[i, sl] = jnp.maximum(hbuf[i, sl] + ebuf[i, sl], 0.0)
            return carry
        lax.fori_loop(0, _C, crow, 0)

    def start_scatter(b):
        pltpu.async_copy(hb[b], agg.at[dib[b]], sss.at[b], add=True)

    def wait_scatter(b):
        pltpu.make_async_copy(hb[b], agg.at[dib[b]], sss.at[b]).wait()

    fetch_ie(0, 0)
    fetch_ie(1, 1)
    fetch_ie(2, 2)
    wait_si(0)
    start_gather(0)

    def quad(i, carry):
        for u in range(4):
            k = 4 * i + u
            b = u
            pb = (u + 3) % 4
            nb = (u + 1) % 4

            @pl.when(jnp.logical_and(k + 3 <= last, k >= 1))
            def _():
                wait_scatter(pb)
                fetch_ie(k + 3, pb)

            @pl.when(jnp.logical_and(k + 3 <= last, k == 0))
            def _():
                fetch_ie(k + 3, pb)

            @pl.when(k + 1 <= last)
            def _():
                wait_si(nb)
                start_gather(nb)

            @pl.when(k <= last)
            def _():
                wait_gather_ed(b)
                compute(b)
                start_scatter(b)
        return carry
    lax.fori_loop(0, (_NCHUNK + 3) // 4, quad, 0)

    for b in range(4):
        wait_scatter(b)
    plsc.subcore_barrier()
    pltpu.sync_copy(agg.at[pl.ds(base, _RPT)],
                    out_hbm.at[c, pl.ds(base, _RPT)])


@functools.cache
def _mp_kernel():
    return functools.partial(
        pl.kernel,
        out_type=jax.ShapeDtypeStruct((_NC, _NPAD, _H), F32),
        mesh=plsc.VectorSubcoreMesh(core_axis_name="c", subcore_axis_name="s",
                                    num_cores=_NC, num_subcores=_NS),
        scratch_types=(
            [pltpu.VMEM((_C, _H), F32)] * 8
            + [pltpu.VMEM((_C,), jnp.int32)] * 8
            + [pltpu.VMEM_SHARED((_NPAD, _H), F32)]
            + [pltpu.SemaphoreType.DMA((4,))] * 5
        ),
    )(_mp_body)


def _mp(h, e, src, dst):
    return _mp_kernel()(h, e, src, dst)


def _enc_body(x_ref, w_ref, b_ref, o_ref):
    o_ref[...] = (
        jnp.dot(x_ref[...], w_ref[...], preferred_element_type=F32)
        + b_ref[...]
    )


def _enc(x, w, b):
    n, d = x.shape
    h = w.shape[1]
    blk = _BLK if n == _N else _EBLK
    return pl.pallas_call(
        _enc_body,
        grid=(n // blk,),
        in_specs=[
            pl.BlockSpec((blk, d), lambda i: (i, 0)),
            pl.BlockSpec((d, h), lambda i: (0, 0)),
            pl.BlockSpec((1, h), lambda i: (0, 0)),
        ],
        out_specs=pl.BlockSpec((blk, h), lambda i: (i, 0)),
        out_shape=jax.ShapeDtypeStruct((n, h), F32),
    )(x, w, b)


def _mlp_body(s_ref, h_ref, a_ref, w1_ref, b1_ref, g1_ref, t1_ref,
              w2_ref, b2_ref, g2_ref, t2_ref, hn_ref, y1v, y2v, st1, st2):
    p = pl.program_id(0)
    j = pl.program_id(1)
    rows = pl.ds(j * _BLK, _BLK)

    @pl.when(p == 0)
    def _():
        @pl.when(j == 0)
        def _():
            st1[...] = jnp.zeros_like(st1)

        z = s_ref[0] * h_ref[...] + a_ref[0] + a_ref[1]
        y = jnp.dot(z, w1_ref[...], preferred_element_type=F32) + b1_ref[...]
        y1v[rows, :] = y
        st1[0:1, :] += jnp.sum(y, axis=0, keepdims=True)
        st1[1:2, :] += jnp.sum(y * y, axis=0, keepdims=True)

    @pl.when(p == 1)
    def _():
        @pl.when(j == 0)
        def _():
            st2[...] = jnp.zeros_like(st2)

        mean = st1[0:1, :] * (1.0 / _N)
        var = st1[1:2, :] * (1.0 / _N) - mean * mean
        sc = lax.rsqrt(var + 1e-5) * g1_ref[...]
        r = jnp.maximum((y1v[rows, :] - mean) * sc + t1_ref[...], 0.0)
        y2 = jnp.dot(r, w2_ref[...], preferred_element_type=F32) + b2_ref[...]
        y2v[rows, :] = y2
        st2[0:1, :] += jnp.sum(y2, axis=0, keepdims=True)
        st2[1:2, :] += jnp.sum(y2 * y2, axis=0, keepdims=True)

    @pl.when(p == 2)
    def _():
        mean = st2[0:1, :] * (1.0 / _N)
        var = st2[1:2, :] * (1.0 / _N) - mean * mean
        sc = lax.rsqrt(var + 1e-5) * g2_ref[...]
        hn_ref[...] = jnp.maximum(
            (y2v[rows, :] - mean) * sc + t2_ref[...], 0.0)


def _mlp_layer(scale, h, agg, w1, b1, g1, t1, w2, b2, g2, t2):
    f1 = w1.shape[1]

    def only_p0(p, j):
        return (jnp.where(p == 0, j, 0), 0)

    def only_p0_3(p, j):
        return (0, jnp.where(p == 0, j, 0), 0)

    return pl.pallas_call(
        _mlp_body,
        grid=(3, _NB),
        in_specs=[
            pl.BlockSpec(memory_space=pltpu.SMEM),
            pl.BlockSpec((_BLK, _H), only_p0),
            pl.BlockSpec((_NC, _BLK, _H), only_p0_3),
            pl.BlockSpec((_H, f1), lambda p, j: (0, 0)),
            pl.BlockSpec((1, f1), lambda p, j: (0, 0)),
            pl.BlockSpec((1, f1), lambda p, j: (0, 0)),
            pl.BlockSpec((1, f1), lambda p, j: (0, 0)),
            pl.BlockSpec((f1, _H), lambda p, j: (0, 0)),
            pl.BlockSpec((1, _H), lambda p, j: (0, 0)),
            pl.BlockSpec((1, _H), lambda p, j: (0, 0)),
            pl.BlockSpec((1, _H), lambda p, j: (0, 0)),
        ],
        out_specs=pl.BlockSpec((_BLK, _H),
                               lambda p, j: (jnp.where(p == 2, j, _NB), 0)),
        out_shape=jax.ShapeDtypeStruct((_N + _BLK, _H), F32),
        scratch_shapes=[
            pltpu.VMEM((_N, f1), F32),
            pltpu.VMEM((_N, _H), F32),
            pltpu.VMEM((8, f1), F32),
            pltpu.VMEM((8, _H), F32),
        ],
    )(scale, h, agg, w1, b1, g1, t1, w2, b2, g2, t2)[:_N]


def _pool_body(b_ref, h0_ref, h1_ref, h2_ref, h3_ref, w1_ref, b1_ref, g_ref,
               bt_ref, wo_ref, bo_ref, o_ref, acc_ref, cnt_ref):
    i = pl.program_id(0)

    @pl.when(i == 0)
    def _():
        acc_ref[...] = jnp.zeros_like(acc_ref)
        cnt_ref[...] = jnp.zeros_like(cnt_ref)

    b = b_ref[0, 0, :]
    oh = (b[:, None] == lax.broadcasted_iota(jnp.int32, (_BLK, _G), 1))
    oh = oh.astype(F32)
    hj = jnp.concatenate(
        [h0_ref[...], h1_ref[...], h2_ref[...], h3_ref[...]], axis=1)
    acc_ref[...] += lax.dot_general(
        oh, hj, (((0,), (0,)), ((), ())), preferred_element_type=F32,
        precision=lax.Precision.HIGHEST)
    cnt_ref[...] += lax.dot_general(
        oh, jnp.ones((_BLK, 8), F32), (((0,), (0,)), ((), ())),
        preferred_element_type=F32, precision=lax.Precision.HIGHEST)

    @pl.when(i == _NB - 1)
    def _():
        cnt = jnp.maximum(cnt_ref[:, 0:1], 1.0)
        pooled = acc_ref[...] / cnt
        o1 = jnp.dot(pooled, w1_ref[...], preferred_element_type=F32)
        o1 = o1 + b1_ref[...]
        mean = jnp.mean(o1, axis=0, keepdims=True)
        var = jnp.mean(o1 * o1, axis=0, keepdims=True) - mean * mean
        scale = lax.rsqrt(var + 1e-5) * g_ref[...]
        r = jnp.maximum((o1 - mean) * scale + bt_ref[...], 0.0)
        o_ref[...] = (
            jnp.dot(r, wo_ref[...], preferred_element_type=F32) + bo_ref[...]
        )


def _pool_cls(batch3, reps, w1, b1, g, bt, wo, bo):
    jk = w1.shape[0]
    ffn = w1.shape[1]
    out = wo.shape[1]
    return pl.pallas_call(
        _pool_body,
        grid=(_NB,),
        in_specs=[
            pl.BlockSpec((1, 1, _BLK), lambda i: (i, 0, 0)),
            pl.BlockSpec((_BLK, _H), lambda i: (i, 0)),
            pl.BlockSpec((_BLK, _H), lambda i: (i, 0)),
            pl.BlockSpec((_BLK, _H), lambda i: (i, 0)),
            pl.BlockSpec((_BLK, _H), lambda i: (i, 0)),
            pl.BlockSpec((jk, ffn), lambda i: (0, 0)),
            pl.BlockSpec((1, ffn), lambda i: (0, 0)),
            pl.BlockSpec((1, ffn), lambda i: (0, 0)),
            pl.BlockSpec((1, ffn), lambda i: (0, 0)),
            pl.BlockSpec((ffn, out), lambda i: (0, 0)),
            pl.BlockSpec((1, out), lambda i: (0, 0)),
        ],
        out_specs=pl.BlockSpec((_G, out), lambda i: (0, 0)),
        out_shape=jax.ShapeDtypeStruct((_G, out), F32),
        scratch_shapes=[
            pltpu.VMEM((_G, jk), F32),
            pltpu.VMEM((_G, 8), F32),
        ],
    )(batch3, *reps, w1, b1, g, bt, wo, bo)


def kernel(x, edge_index, edge_attr, batch, params):
    src = edge_index[0]
    dst = edge_index[1]

    h = _enc(x, params['enc_W'], params['enc_b'].reshape(1, -1))
    e = _enc(edge_attr, params['edge_W'], params['edge_b'].reshape(1, -1))

    reps = [h]
    for l in range(3):
        agg = _mp(h, e, src, dst)
        scale = (1.0 + params[f'conv{l}_eps']).reshape(1)
        h = _mlp_layer(scale, h, agg,
                       params[f'conv{l}_W1'], params[f'conv{l}_b1'].reshape(1, -1),
                       params[f'conv{l}_g1'].reshape(1, -1), params[f'conv{l}_bt1'].reshape(1, -1),
                       params[f'conv{l}_W2'], params[f'conv{l}_b2'].reshape(1, -1),
                       params[f'conv{l}_g2'].reshape(1, -1), params[f'conv{l}_bt2'].reshape(1, -1))
        reps.append(h)

    batch3 = batch.reshape(_NB, 1, _BLK)
    return _pool_cls(batch3, reps, params['cls_W1'],
                     params['cls_b1'].reshape(1, -1),
                     params['cls_g'].reshape(1, -1),
                     params['cls_bt'].reshape(1, -1),
                     params['out_W'], params['out_b'].reshape(1, -1))
